# baseline (speedup 1.0000x reference)
import functools

import jax
import jax.numpy as jnp
from jax import lax
from jax.experimental import pallas as pl
from jax.experimental.pallas import tpu as pltpu
from jax.experimental.pallas import tpu_sc as plsc

BATCH = 16384
EMBED = 64
IDX_MINOR = 128
L = 16


def _sc_kernel(head2, tail2, table, w_t, b16, out0, out1,
               hidx, tidx, hrows, trows, wv, bv, o0v, o1v, sem):
    nc = 2
    wid = lax.axis_index("s") * nc + lax.axis_index("c")
    b_per_w = BATCH // 32
    n_chunks = b_per_w // IDX_MINOR

    pltpu.sync_copy(head2.at[pl.ds(wid * n_chunks, n_chunks)], hidx)
    pltpu.sync_copy(tail2.at[pl.ds(wid * n_chunks, n_chunks)], tidx)
    pltpu.sync_copy(w_t, wv)
    pltpu.sync_copy(b16, bv)

    copies = []
    for k in range(n_chunks):
        copies.append(pltpu.async_copy(
            table.at[hidx.at[k]], hrows.at[pl.ds(IDX_MINOR * k, IDX_MINOR)], sem))
    for k in range(n_chunks):
        copies.append(pltpu.async_copy(
            table.at[tidx.at[k]], trows.at[pl.ds(IDX_MINOR * k, IDX_MINOR)], sem))
    for c in copies:
        c.wait()

    w0 = [wv[0, pl.ds(L * k, L)] for k in range(8)]
    w1 = [wv[1, pl.ds(L * k, L)] for k in range(8)]
    binit0 = bv[0, pl.ds(0, L)]
    binit1 = bv[1, pl.ds(0, L)]
    lastmask = lax.iota(jnp.int32, L) == (L - 1)

    def body(i, carry):
        acc0 = binit0
        acc1 = binit1
        for k in range(4):
            hk = hrows[i, pl.ds(L * k, L)]
            acc0 = acc0 + hk * w0[k]
            acc1 = acc1 + hk * w1[k]
        for k in range(4):
            tk = trows[i, pl.ds(L * k, L)]
            acc0 = acc0 + tk * w0[4 + k]
            acc1 = acc1 + tk * w1[4 + k]
        s0 = plsc.cumsum(acc0)
        s1 = plsc.cumsum(acc1)
        idxv = jnp.zeros((L,), jnp.int32) + i
        plsc.store_scatter(o0v, [idxv], s0, mask=lastmask)
        plsc.store_scatter(o1v, [idxv], s1, mask=lastmask)
        return carry

    lax.fori_loop(0, b_per_w, body, 0)

    pltpu.sync_copy(o0v, out0.at[pl.ds(wid * b_per_w, b_per_w)])
    pltpu.sync_copy(o1v, out1.at[pl.ds(wid * b_per_w, b_per_w)])


def kernel(head, tail, table, fc_w, fc_b):
    b_per_w = BATCH // 32
    head2 = head.reshape(BATCH // IDX_MINOR, IDX_MINOR)
    tail2 = tail.reshape(BATCH // IDX_MINOR, IDX_MINOR)
    w_t = fc_w.T
    b16 = jnp.zeros((2, L), jnp.float32).at[:, 0].set(fc_b)

    mesh = plsc.VectorSubcoreMesh(core_axis_name="c", subcore_axis_name="s")
    n_chunks = b_per_w // IDX_MINOR
    run = pl.kernel(
        _sc_kernel,
        mesh=mesh,
        compiler_params=pltpu.CompilerParams(
            needs_layout_passes=False, use_tc_tiling_on_sc=False),
        out_type=[
            jax.ShapeDtypeStruct((BATCH,), jnp.float32),
            jax.ShapeDtypeStruct((BATCH,), jnp.float32),
        ],
        scratch_types=[
            pltpu.VMEM((n_chunks, IDX_MINOR), jnp.int32),
            pltpu.VMEM((n_chunks, IDX_MINOR), jnp.int32),
            pltpu.VMEM((b_per_w, EMBED), jnp.float32),
            pltpu.VMEM((b_per_w, EMBED), jnp.float32),
            pltpu.VMEM((2, 2 * EMBED), jnp.float32),
            pltpu.VMEM((2, L), jnp.float32),
            pltpu.VMEM((b_per_w,), jnp.float32),
            pltpu.VMEM((b_per_w,), jnp.float32),
            pltpu.SemaphoreType.DMA,
        ],
    )
    o0, o1 = run(head2, tail2, table, w_t, b16)
    return jnp.stack([o0, o1], axis=1)
